# baseline (speedup 1.0000x reference)
import functools

import jax
import jax.numpy as jnp
from jax import lax
from jax.experimental import pallas as pl
from jax.experimental.pallas import tpu as pltpu
from jax.experimental.pallas import tpu_sc as plsc

VOCAB = 1000000
EMBED = 64
UNIT = 64
B = 4096
T = 50

NC = 2
NS = 16
NW = NC * NS
CHUNK = 128

SEQ_PER_W = (B * T) // NW
SEQ_CHUNKS = SEQ_PER_W // CHUNK
POS_PER_W = B // NW


def _sc_gather(table, seq_idx, pos_idx, neg_idx):
    mesh = plsc.VectorSubcoreMesh(core_axis_name="c", subcore_axis_name="s")

    @functools.partial(
        pl.kernel,
        mesh=mesh,
        out_type=(
            jax.ShapeDtypeStruct((B * T, EMBED), jnp.float32),
            jax.ShapeDtypeStruct((B, EMBED), jnp.float32),
            jax.ShapeDtypeStruct((B, EMBED), jnp.float32),
        ),
        scratch_types=[
            pltpu.VMEM((CHUNK,), jnp.int32),
            pltpu.VMEM((CHUNK, EMBED), jnp.float32),
            pltpu.SemaphoreType.DMA,
        ],
        compiler_params=pltpu.CompilerParams(use_tc_tiling_on_sc=False),
    )
    def k(table_hbm, seq_hbm, pos_hbm, neg_hbm, out_seq, out_pos, out_neg,
          idx_v, rows_v, sem):
        wid = lax.axis_index("s") * NC + lax.axis_index("c")

        def gather_chunk(idx_hbm, out_hbm, off):
            pltpu.sync_copy(idx_hbm.at[pl.ds(off, CHUNK)], idx_v)
            pltpu.async_copy(table_hbm.at[idx_v], rows_v, sem).wait()
            pltpu.sync_copy(rows_v, out_hbm.at[pl.ds(off, CHUNK)])

        seq_base = wid * SEQ_PER_W

        def body(c, carry):
            gather_chunk(seq_hbm, out_seq, seq_base + c * CHUNK)
            return carry

        lax.fori_loop(0, SEQ_CHUNKS, body, 0)
        gather_chunk(pos_hbm, out_pos, wid * POS_PER_W)
        gather_chunk(neg_hbm, out_neg, wid * POS_PER_W)

    return k(table, seq_idx, pos_idx, neg_idx)


BLK = 256


def _tc_gru(seq_e, ids, pos_e, neg_e, W, U, b, Wd, bd2):
    G = B // BLK

    def body(seq_ref, ids_ref, pos_ref, neg_ref, W_ref, U_ref, b_ref,
             Wd_ref, bd_ref, out_ref):
        Wm = W_ref[...]
        Um = U_ref[...]
        b0 = b_ref[0:1, :]
        b1 = b_ref[1:2, :]
        mask = (ids_ref[...] != 0).astype(jnp.float32)
        h = jnp.zeros((BLK, UNIT), jnp.float32)
        for t in range(T):
            xt = seq_ref[:, t, :] * mask[:, t:t + 1]
            mx = jnp.dot(xt, Wm, preferred_element_type=jnp.float32) + b0
            mi = jnp.dot(h, Um, preferred_element_type=jnp.float32) + b1
            z = jax.nn.sigmoid(mx[:, :UNIT] + mi[:, :UNIT])
            r = jax.nn.sigmoid(mx[:, UNIT:2 * UNIT] + mi[:, UNIT:2 * UNIT])
            hh = jnp.tanh(mx[:, 2 * UNIT:] + r * mi[:, 2 * UNIT:])
            h = z * h + (1.0 - z) * hh
        s = jnp.dot(h, Wd_ref[...], preferred_element_type=jnp.float32) + bd_ref[...]

        def l2n(v):
            return v * lax.rsqrt(jnp.maximum(jnp.sum(v * v, axis=-1, keepdims=True), 1e-12))

        s = l2n(s)
        p = l2n(pos_ref[...])
        q = l2n(neg_ref[...])
        ps = jnp.sum(s * p, axis=-1, keepdims=True)
        ns = jnp.sum(s * q, axis=-1, keepdims=True)
        out_ref[...] = jnp.concatenate([ps, ns], axis=1)

    return pl.pallas_call(
        body,
        grid=(G,),
        in_specs=[
            pl.BlockSpec((BLK, T, EMBED), lambda i: (i, 0, 0)),
            pl.BlockSpec((BLK, T), lambda i: (i, 0)),
            pl.BlockSpec((BLK, EMBED), lambda i: (i, 0)),
            pl.BlockSpec((BLK, EMBED), lambda i: (i, 0)),
            pl.BlockSpec((EMBED, 3 * UNIT), lambda i: (0, 0)),
            pl.BlockSpec((UNIT, 3 * UNIT), lambda i: (0, 0)),
            pl.BlockSpec((2, 3 * UNIT), lambda i: (0, 0)),
            pl.BlockSpec((UNIT, EMBED), lambda i: (0, 0)),
            pl.BlockSpec((1, EMBED), lambda i: (0, 0)),
        ],
        out_specs=pl.BlockSpec((BLK, 2), lambda i: (i, 0)),
        out_shape=jax.ShapeDtypeStruct((B, 2), jnp.float32),
    )(seq_e, ids, pos_e, neg_e, W, U, b, Wd, bd2)


def kernel(click_seq, pos_item, neg_item, table, W, U, b, Wd, bd):
    click_seq = click_seq.astype(jnp.int32)
    seq_idx = click_seq.reshape(-1)
    pos_idx = pos_item.astype(jnp.int32).reshape(-1)
    neg_idx = neg_item.astype(jnp.int32).reshape(-1)
    seq_e, pos_e, neg_e = _sc_gather(table, seq_idx, pos_idx, neg_idx)
    seq_e = seq_e.reshape(B, T, EMBED)
    return _tc_gru(seq_e, click_seq, pos_e, neg_e, W, U, b, Wd,
                   bd.reshape(1, EMBED))


# speedup vs baseline: 1.2436x; 1.2436x over previous
import functools

import jax
import jax.numpy as jnp
from jax import lax
from jax.experimental import pallas as pl
from jax.experimental.pallas import tpu as pltpu
from jax.experimental.pallas import tpu_sc as plsc

VOCAB = 1000000
EMBED = 64
UNIT = 64
B = 4096
T = 50
LANE = 2 * EMBED

NC = 2
NS = 16
NW = NC * NS
CHUNK = 128

SEQ_ROWS = B * T
SEQ_PER_W = SEQ_ROWS // NW
SEQ_CHUNKS = SEQ_PER_W // CHUNK
POS_PER_W = B // NW


def _sc_gather(table, seq_idx, pos_idx, neg_idx):
    mesh = plsc.VectorSubcoreMesh(core_axis_name="c", subcore_axis_name="s")

    @functools.partial(
        pl.kernel,
        mesh=mesh,
        out_type=(
            jax.ShapeDtypeStruct((T * B, LANE), jnp.float32),
            jax.ShapeDtypeStruct((B, LANE), jnp.float32),
            jax.ShapeDtypeStruct((B, LANE), jnp.float32),
        ),
        scratch_types=[
            pltpu.VMEM((CHUNK,), jnp.int32),
            pltpu.VMEM((CHUNK, EMBED), jnp.float32),
            pltpu.SemaphoreType.DMA,
        ],
        compiler_params=pltpu.CompilerParams(use_tc_tiling_on_sc=False),
    )
    def k(table_hbm, seq_hbm, pos_hbm, neg_hbm, out_seq, out_pos, out_neg,
          idx_v, rows_v, sem):
        wid = lax.axis_index("s") * NC + lax.axis_index("c")
        oseq = out_seq

        def gather_chunk(idx_hbm, out_view, off):
            pltpu.sync_copy(idx_hbm.at[pl.ds(off, CHUNK)], idx_v)
            pltpu.async_copy(table_hbm.at[idx_v], rows_v, sem).wait()
            pltpu.sync_copy(rows_v, out_view.at[pl.ds(off, CHUNK), pl.ds(0, EMBED)])

        seq_base = wid * SEQ_PER_W

        def body(c, carry):
            gather_chunk(seq_hbm, oseq, seq_base + c * CHUNK)
            return carry

        lax.fori_loop(0, SEQ_CHUNKS, body, 0)
        gather_chunk(pos_hbm, out_pos, wid * POS_PER_W)
        gather_chunk(neg_hbm, out_neg, wid * POS_PER_W)

    return k(table, seq_idx, pos_idx, neg_idx)


BLK = 512


def _tc_gru(seq_e, ids, pos_e, neg_e, W, U, b, Wd, bd2):
    G = B // BLK

    def body(seq_ref, ids_ref, pos_ref, neg_ref, W_ref, U_ref, b_ref,
             Wd_ref, bd_ref, out_ref):
        Wm = W_ref[...]
        Um = U_ref[...]
        b0 = b_ref[0:1, :]
        b1 = b_ref[1:2, :]
        mask = (ids_ref[...] != 0).astype(jnp.float32)
        h = jnp.zeros((BLK, UNIT), jnp.float32)
        for t in range(T):
            xt = seq_ref[t][:, :EMBED] * mask[:, t:t + 1]
            mx = jnp.dot(xt, Wm, preferred_element_type=jnp.float32) + b0
            mi = jnp.dot(h, Um, preferred_element_type=jnp.float32) + b1
            zr = jax.nn.sigmoid(mx[:, :2 * UNIT] + mi[:, :2 * UNIT])
            z = zr[:, :UNIT]
            r = zr[:, UNIT:]
            hh = jnp.tanh(mx[:, 2 * UNIT:] + r * mi[:, 2 * UNIT:])
            h = z * h + (1.0 - z) * hh
        s = jnp.dot(h, Wd_ref[...], preferred_element_type=jnp.float32) + bd_ref[...]

        def l2n(v):
            return v * lax.rsqrt(jnp.maximum(jnp.sum(v * v, axis=-1, keepdims=True), 1e-12))

        s = l2n(s)
        p = l2n(pos_ref[...][:, :EMBED])
        q = l2n(neg_ref[...][:, :EMBED])
        ps = jnp.sum(s * p, axis=-1, keepdims=True)
        ns = jnp.sum(s * q, axis=-1, keepdims=True)
        out_ref[...] = jnp.concatenate([ps, ns], axis=1)

    return pl.pallas_call(
        body,
        grid=(G,),
        in_specs=[
            pl.BlockSpec((T, BLK, LANE), lambda i: (0, i, 0)),
            pl.BlockSpec((BLK, T), lambda i: (i, 0)),
            pl.BlockSpec((BLK, LANE), lambda i: (i, 0)),
            pl.BlockSpec((BLK, LANE), lambda i: (i, 0)),
            pl.BlockSpec((EMBED, 3 * UNIT), lambda i: (0, 0)),
            pl.BlockSpec((UNIT, 3 * UNIT), lambda i: (0, 0)),
            pl.BlockSpec((2, 3 * UNIT), lambda i: (0, 0)),
            pl.BlockSpec((UNIT, EMBED), lambda i: (0, 0)),
            pl.BlockSpec((1, EMBED), lambda i: (0, 0)),
        ],
        out_specs=pl.BlockSpec((BLK, 2), lambda i: (i, 0)),
        out_shape=jax.ShapeDtypeStruct((B, 2), jnp.float32),
    )(seq_e, ids, pos_e, neg_e, W, U, b, Wd, bd2)


def kernel(click_seq, pos_item, neg_item, table, W, U, b, Wd, bd):
    click_seq = click_seq.astype(jnp.int32)
    seq_idx = click_seq.T.reshape(-1)
    pos_idx = pos_item.astype(jnp.int32).reshape(-1)
    neg_idx = neg_item.astype(jnp.int32).reshape(-1)
    seq_e, pos_e, neg_e = _sc_gather(table, seq_idx, pos_idx, neg_idx)
    seq_e = seq_e.reshape(T, B, LANE)
    return _tc_gru(seq_e, click_seq, pos_e, neg_e, W, U, b, Wd,
                   bd.reshape(1, EMBED))


# speedup vs baseline: 1.3176x; 1.0595x over previous
import functools

import jax
import jax.numpy as jnp
from jax import lax
from jax.experimental import pallas as pl
from jax.experimental.pallas import tpu as pltpu
from jax.experimental.pallas import tpu_sc as plsc

VOCAB = 1000000
EMBED = 64
UNIT = 64
B = 4096
T = 50
LANE = 2 * EMBED

NC = 2
NS = 16
NW = NC * NS
CHUNK = 128

def _sc_gather(table, seq_idx, pos_idx, neg_idx, nb):
    mesh = plsc.VectorSubcoreMesh(core_axis_name="c", subcore_axis_name="s")
    seq_per_w = (nb * T) // NW
    seq_chunks = seq_per_w // CHUNK
    seq_rem = seq_per_w % CHUNK
    pn_per_w = nb // NW

    @functools.partial(
        pl.kernel,
        mesh=mesh,
        out_type=(
            jax.ShapeDtypeStruct((T * nb, LANE), jnp.float32),
            jax.ShapeDtypeStruct((nb, LANE), jnp.float32),
            jax.ShapeDtypeStruct((nb, LANE), jnp.float32),
        ),
        scratch_types=[
            pltpu.VMEM((CHUNK,), jnp.int32),
            pltpu.VMEM((CHUNK, EMBED), jnp.float32),
            pltpu.SemaphoreType.DMA,
        ],
        compiler_params=pltpu.CompilerParams(use_tc_tiling_on_sc=False),
    )
    def k(table_hbm, seq_hbm, pos_hbm, neg_hbm, out_seq, out_pos, out_neg,
          idx_v, rows_v, sem):
        wid = lax.axis_index("s") * NC + lax.axis_index("c")

        def gather_chunk(idx_hbm, out_view, off, n):
            pltpu.sync_copy(idx_hbm.at[pl.ds(off, n)], idx_v.at[pl.ds(0, n)])
            pltpu.async_copy(table_hbm.at[idx_v.at[pl.ds(0, n)]],
                             rows_v.at[pl.ds(0, n)], sem).wait()
            pltpu.sync_copy(rows_v.at[pl.ds(0, n)],
                            out_view.at[pl.ds(off, n), pl.ds(0, EMBED)])

        seq_base = wid * seq_per_w

        def body(c, carry):
            gather_chunk(seq_hbm, out_seq, seq_base + c * CHUNK, CHUNK)
            return carry

        lax.fori_loop(0, seq_chunks, body, 0)
        if seq_rem:
            gather_chunk(seq_hbm, out_seq, seq_base + seq_chunks * CHUNK, seq_rem)
        gather_chunk(pos_hbm, out_pos, wid * pn_per_w, pn_per_w)
        gather_chunk(neg_hbm, out_neg, wid * pn_per_w, pn_per_w)

    return k(table, seq_idx, pos_idx, neg_idx)


BLK = 512


def _tc_gru(seq_e, ids, pos_e, neg_e, W, U, b, Wd, bd2, nb):
    G = nb // BLK

    def body(seq_ref, ids_ref, pos_ref, neg_ref, W_ref, U_ref, b_ref,
             Wd_ref, bd_ref, out_ref):
        Wm = W_ref[...]
        Um = U_ref[...]
        b0 = b_ref[0:1, :]
        b1 = b_ref[1:2, :]
        mask = (ids_ref[...] != 0).astype(jnp.float32)
        h = jnp.zeros((BLK, UNIT), jnp.float32)
        for t in range(T):
            xt = seq_ref[t][:, :EMBED] * mask[:, t:t + 1]
            mx = jnp.dot(xt, Wm, preferred_element_type=jnp.float32) + b0
            mi = jnp.dot(h, Um, preferred_element_type=jnp.float32) + b1
            zr = 0.5 * jnp.tanh(0.5 * (mx[:, :2 * UNIT] + mi[:, :2 * UNIT])) + 0.5
            z = zr[:, :UNIT]
            r = zr[:, UNIT:]
            hh = jnp.tanh(mx[:, 2 * UNIT:] + r * mi[:, 2 * UNIT:])
            h = z * h + (1.0 - z) * hh
        s = jnp.dot(h, Wd_ref[...], preferred_element_type=jnp.float32) + bd_ref[...]

        def l2n(v):
            return v * lax.rsqrt(jnp.maximum(jnp.sum(v * v, axis=-1, keepdims=True), 1e-12))

        s = l2n(s)
        p = l2n(pos_ref[...][:, :EMBED])
        q = l2n(neg_ref[...][:, :EMBED])
        ps = jnp.sum(s * p, axis=-1, keepdims=True)
        ns = jnp.sum(s * q, axis=-1, keepdims=True)
        out_ref[...] = jnp.concatenate([ps, ns], axis=1)

    return pl.pallas_call(
        body,
        grid=(G,),
        in_specs=[
            pl.BlockSpec((T, BLK, LANE), lambda i: (0, i, 0)),
            pl.BlockSpec((BLK, T), lambda i: (i, 0)),
            pl.BlockSpec((BLK, LANE), lambda i: (i, 0)),
            pl.BlockSpec((BLK, LANE), lambda i: (i, 0)),
            pl.BlockSpec((EMBED, 3 * UNIT), lambda i: (0, 0)),
            pl.BlockSpec((UNIT, 3 * UNIT), lambda i: (0, 0)),
            pl.BlockSpec((2, 3 * UNIT), lambda i: (0, 0)),
            pl.BlockSpec((UNIT, EMBED), lambda i: (0, 0)),
            pl.BlockSpec((1, EMBED), lambda i: (0, 0)),
        ],
        out_specs=pl.BlockSpec((BLK, 2), lambda i: (i, 0)),
        out_shape=jax.ShapeDtypeStruct((nb, 2), jnp.float32),
    )(seq_e, ids, pos_e, neg_e, W, U, b, Wd, bd2)


HALVES = 4
NB = B // HALVES


def kernel(click_seq, pos_item, neg_item, table, W, U, b, Wd, bd):
    click_seq = click_seq.astype(jnp.int32)
    pos_flat = pos_item.astype(jnp.int32).reshape(-1)
    neg_flat = neg_item.astype(jnp.int32).reshape(-1)
    bd2 = bd.reshape(1, EMBED)
    outs = []
    for h in range(HALVES):
        cs_h = click_seq[h * NB:(h + 1) * NB]
        seq_idx = cs_h.T.reshape(-1)
        pos_idx = pos_flat[h * NB:(h + 1) * NB]
        neg_idx = neg_flat[h * NB:(h + 1) * NB]
        seq_e, pos_e, neg_e = _sc_gather(table, seq_idx, pos_idx, neg_idx, NB)
        seq_e = seq_e.reshape(T, NB, LANE)
        outs.append(_tc_gru(seq_e, cs_h, pos_e, neg_e, W, U, b, Wd, bd2, NB))
    return jnp.concatenate(outs, axis=0)
